# Initial kernel scaffold; baseline (speedup 1.0000x reference)
import jax
import jax.numpy as jnp
from jax.experimental import pallas as pl


def kernel(x, emb_table, fc_table, bias, W0, b0, W1, b1, W2, b2, W3, b3):
    raise NotImplementedError("write your pallas kernel here")


# speedup vs baseline: 1.3233x; 1.3233x over previous
import functools

import jax
import jax.numpy as jnp
from jax import lax
from jax.experimental import pallas as pl
from jax.experimental.pallas import tpu as pltpu
from jax.experimental.pallas import tpu_sc as plsc

V = 1000012
D = 16
F = 26
B = 16384
BF = B * F

NC = 2
NS = 16
NW = NC * NS
PER_W = BF // NW
CHUNK = 1664
N_CHUNKS = PER_W // CHUNK


def _sc_gather_body(emb_hbm, fc_hbm, idx_hbm, out_emb, out_fc,
                    idx_v, rows_v, fcrows_v, sem_e, sem_f):
    wid = lax.axis_index("s") * NC + lax.axis_index("c")
    base = wid * PER_W
    for g in range(N_CHUNKS):
        off = base + g * CHUNK
        pltpu.sync_copy(idx_hbm.at[pl.ds(off, CHUNK)], idx_v)
        cp_e = pltpu.async_copy(emb_hbm.at[idx_v], rows_v, sem_e)
        cp_f = pltpu.async_copy(fc_hbm.at[idx_v], fcrows_v, sem_f)
        cp_e.wait()
        cp_f.wait()
        pltpu.sync_copy(rows_v, out_emb.at[pl.ds(off, CHUNK)])
        pltpu.sync_copy(fcrows_v, out_fc.at[pl.ds(off, CHUNK)])


@jax.jit
def _sc_gather(emb_table, fc_table, idx):
    mesh = plsc.VectorSubcoreMesh(core_axis_name="c", subcore_axis_name="s")
    return pl.kernel(
        _sc_gather_body,
        out_type=(
            jax.ShapeDtypeStruct((BF, D), jnp.float32),
            jax.ShapeDtypeStruct((BF,), jnp.float32),
        ),
        mesh=mesh,
        scratch_types=[
            pltpu.VMEM((CHUNK,), jnp.int32),
            pltpu.VMEM((CHUNK, D), jnp.float32),
            pltpu.VMEM((CHUNK,), jnp.float32),
            pltpu.SemaphoreType.DMA,
            pltpu.SemaphoreType.DMA,
        ],
        compiler_params=pltpu.CompilerParams(use_tc_tiling_on_sc=False),
    )(emb_table, fc_table, idx)


BLK = 1024


def _tc_body(e_ref, fcg_ref, scal_ref, w0_ref, b0_ref, w1_ref, b1_ref,
             w2_ref, b2_ref, w3_ref, out_ref):
    e = e_ref[...]
    r = lax.broadcasted_iota(jnp.int32, (F * D, D), 0)
    c = lax.broadcasted_iota(jnp.int32, (F * D, D), 1)
    s_mat = jnp.where((r % D) == c, 1.0, 0.0).astype(jnp.float32)
    sums = jnp.dot(e, s_mat, preferred_element_type=jnp.float32)
    sqs = jnp.dot(e * e, s_mat, preferred_element_type=jnp.float32)
    inter = 0.5 * jnp.sum(sums * sums - sqs, axis=1, keepdims=True)
    fc_sum = jnp.sum(fcg_ref[...], axis=1, keepdims=True)
    h = jnp.maximum(jnp.dot(e, w0_ref[...], preferred_element_type=jnp.float32)
                    + b0_ref[...], 0.0)
    h = jnp.maximum(jnp.dot(h, w1_ref[...], preferred_element_type=jnp.float32)
                    + b1_ref[...], 0.0)
    h = jnp.maximum(jnp.dot(h, w2_ref[...], preferred_element_type=jnp.float32)
                    + b2_ref[...], 0.0)
    mlp = jnp.sum(h * w3_ref[...], axis=1, keepdims=True)
    z = inter + fc_sum + mlp + scal_ref[0]
    out_ref[...] = 1.0 / (1.0 + jnp.exp(-z))


@jax.jit
def _tc_head(e_mat, fc_mat, scal, w0, b0, w1, b1, w2, b2, w3t):
    grid = (B // BLK,)
    return pl.pallas_call(
        _tc_body,
        grid=grid,
        in_specs=[
            pl.BlockSpec((BLK, F * D), lambda i: (i, 0)),
            pl.BlockSpec((BLK, F), lambda i: (i, 0)),
            pl.BlockSpec(memory_space=pltpu.SMEM),
            pl.BlockSpec((F * D, 256), lambda i: (0, 0)),
            pl.BlockSpec((1, 256), lambda i: (0, 0)),
            pl.BlockSpec((256, 128), lambda i: (0, 0)),
            pl.BlockSpec((1, 128), lambda i: (0, 0)),
            pl.BlockSpec((128, 64), lambda i: (0, 0)),
            pl.BlockSpec((1, 64), lambda i: (0, 0)),
            pl.BlockSpec((1, 64), lambda i: (0, 0)),
        ],
        out_specs=pl.BlockSpec((BLK, 1), lambda i: (i, 0)),
        out_shape=jax.ShapeDtypeStruct((B, 1), jnp.float32),
    )(e_mat, fc_mat, scal, w0, b0, w1, b1, w2, b2, w3t)


def kernel(x, emb_table, fc_table, bias, W0, b0, W1, b1, W2, b2, W3, b3):
    idx = x.reshape(-1).astype(jnp.int32)
    e_flat, fc_flat = _sc_gather(emb_table, fc_table.reshape(V), idx)
    e_mat = e_flat.reshape(B, F * D)
    fc_mat = fc_flat.reshape(B, F)
    scal = (bias + b3).astype(jnp.float32)
    out = _tc_head(e_mat, fc_mat, scal,
                   W0, b0.reshape(1, 256), W1, b1.reshape(1, 128),
                   W2, b2.reshape(1, 64), W3.reshape(1, 64))
    return out.reshape(B)


# speedup vs baseline: 1.5198x; 1.1485x over previous
import functools

import jax
import jax.numpy as jnp
from jax import lax
from jax.experimental import pallas as pl
from jax.experimental.pallas import tpu as pltpu
from jax.experimental.pallas import tpu_sc as plsc

V = 1000012
D = 16
F = 26
B = 16384
BF = B * F

VB = 16384
VZ = ((V + VB - 1) // VB) * VB

NC = 2
NS = 16
NW = NC * NS
PER_W = BF // NW
CHUNK = 1664
N_CHUNKS = PER_W // CHUNK


def _lin_body(t_ref, out_ref):
    i = pl.program_id(0)
    e = t_ref[...]
    col = lax.broadcasted_iota(jnp.int32, (D, VB), 1) + i * VB
    e = jnp.where(col < V, e, 0.0)
    g = e.T.reshape(VB // 8, 8, D)
    out_ref[...] = jnp.concatenate([g[:, j, :] for j in range(8)], axis=1)


def _linearize(emb_t):
    return pl.pallas_call(
        _lin_body,
        grid=(VZ // VB,),
        in_specs=[pl.BlockSpec((D, VB), lambda i: (0, i))],
        out_specs=pl.BlockSpec((VB // 8, 128), lambda i: (i, 0)),
        out_shape=jax.ShapeDtypeStruct((VZ // 8, 128), jnp.float32),
    )(emb_t)


def _sc_gather_body(emb_hbm, fc_hbm, idx_hbm, out_emb, out_fc,
                    idx_v, rows_v, fcrows_v, sem_e, sem_f):
    wid = lax.axis_index("s") * NC + lax.axis_index("c")
    base = wid * PER_W
    for g in range(N_CHUNKS):
        off = base + g * CHUNK
        pltpu.sync_copy(idx_hbm.at[pl.ds(off, CHUNK)], idx_v)
        cp_e = pltpu.async_copy(emb_hbm.at[idx_v], rows_v, sem_e)
        cp_f = pltpu.async_copy(fc_hbm.at[idx_v], fcrows_v, sem_f)
        cp_e.wait()
        cp_f.wait()
        pltpu.sync_copy(rows_v, out_emb.at[pl.ds(off, CHUNK)])
        pltpu.sync_copy(fcrows_v, out_fc.at[pl.ds(off, CHUNK)])


def _sc_gather(emb_lin, fc_flat, idx):
    mesh = plsc.VectorSubcoreMesh(core_axis_name="c", subcore_axis_name="s")
    return pl.kernel(
        _sc_gather_body,
        out_type=(
            jax.ShapeDtypeStruct((BF, D), jnp.float32),
            jax.ShapeDtypeStruct((BF,), jnp.float32),
        ),
        mesh=mesh,
        scratch_types=[
            pltpu.VMEM((CHUNK,), jnp.int32),
            pltpu.VMEM((CHUNK, D), jnp.float32),
            pltpu.VMEM((CHUNK,), jnp.float32),
            pltpu.SemaphoreType.DMA,
            pltpu.SemaphoreType.DMA,
        ],
        compiler_params=pltpu.CompilerParams(use_tc_tiling_on_sc=False),
    )(emb_lin, fc_flat, idx)


BLK = 1024


def _tc_body(e_ref, fcg_ref, scal_ref, w0_ref, b0_ref, w1_ref, b1_ref,
             w2_ref, b2_ref, w3_ref, out_ref):
    e = e_ref[...]
    r = lax.broadcasted_iota(jnp.int32, (F * D, D), 0)
    c = lax.broadcasted_iota(jnp.int32, (F * D, D), 1)
    s_mat = jnp.where((r % D) == c, 1.0, 0.0).astype(jnp.float32)
    sums = jnp.dot(e, s_mat, preferred_element_type=jnp.float32)
    sqs = jnp.dot(e * e, s_mat, preferred_element_type=jnp.float32)
    inter = 0.5 * jnp.sum(sums * sums - sqs, axis=1, keepdims=True)
    fc_sum = jnp.sum(fcg_ref[...], axis=1, keepdims=True)
    h = jnp.maximum(jnp.dot(e, w0_ref[...], preferred_element_type=jnp.float32)
                    + b0_ref[...], 0.0)
    h = jnp.maximum(jnp.dot(h, w1_ref[...], preferred_element_type=jnp.float32)
                    + b1_ref[...], 0.0)
    h = jnp.maximum(jnp.dot(h, w2_ref[...], preferred_element_type=jnp.float32)
                    + b2_ref[...], 0.0)
    mlp = jnp.sum(h * w3_ref[...], axis=1, keepdims=True)
    z = inter + fc_sum + mlp + scal_ref[0]
    out_ref[...] = 1.0 / (1.0 + jnp.exp(-z))


def _tc_head(e_mat, fc_mat, scal, w0, b0, w1, b1, w2, b2, w3t):
    grid = (B // BLK,)
    return pl.pallas_call(
        _tc_body,
        grid=grid,
        in_specs=[
            pl.BlockSpec((BLK, F * D), lambda i: (i, 0)),
            pl.BlockSpec((BLK, F), lambda i: (i, 0)),
            pl.BlockSpec(memory_space=pltpu.SMEM),
            pl.BlockSpec((F * D, 256), lambda i: (0, 0)),
            pl.BlockSpec((1, 256), lambda i: (0, 0)),
            pl.BlockSpec((256, 128), lambda i: (0, 0)),
            pl.BlockSpec((1, 128), lambda i: (0, 0)),
            pl.BlockSpec((128, 64), lambda i: (0, 0)),
            pl.BlockSpec((1, 64), lambda i: (0, 0)),
            pl.BlockSpec((1, 64), lambda i: (0, 0)),
        ],
        out_specs=pl.BlockSpec((BLK, 1), lambda i: (i, 0)),
        out_shape=jax.ShapeDtypeStruct((B, 1), jnp.float32),
    )(e_mat, fc_mat, scal, w0, b0, w1, b1, w2, b2, w3t)


def kernel(x, emb_table, fc_table, bias, W0, b0, W1, b1, W2, b2, W3, b3):
    idx = x.reshape(-1).astype(jnp.int32)
    emb_lin = _linearize(emb_table.T).reshape(VZ, D)
    e_flat, fc_flat = _sc_gather(emb_lin, fc_table.reshape(V), idx)
    e_mat = e_flat.reshape(B, F * D)
    fc_mat = fc_flat.reshape(B, F)
    scal = (bias + b3).astype(jnp.float32)
    out = _tc_head(e_mat, fc_mat, scal,
                   W0, b0.reshape(1, 256), W1, b1.reshape(1, 128),
                   W2, b2.reshape(1, 64), W3.reshape(1, 64))
    return out.reshape(B)
